# Initial kernel scaffold; baseline (speedup 1.0000x reference)
import functools

import jax
import jax.numpy as jnp
from jax import lax
from jax.experimental import pallas as pl
from jax.experimental.pallas import tpu as pltpu
from jax.experimental.pallas import tpu_sc as plsc

N = 10000
E = 320000
D = 128
H = 64
OUT = 64
E_POS = E + N
NPAIR = 2 * E_POS

NC = 2
NS = 16
NW = NC * NS
EPT = E // NW
ROWS_PER_SUB = N // NS

EW = 80
N_EWIN = EPT // EW

PW = 128
PPT = 20736
N_PWIN = PPT // PW
NPAIR_PAD = NW * PPT

_mesh = plsc.VectorSubcoreMesh(core_axis_name="c", subcore_axis_name="s")


def _wid():
    return lax.axis_index("s") * NC + lax.axis_index("c")


@functools.partial(
    pl.kernel,
    out_type=jax.ShapeDtypeStruct((NW, N), jnp.float32),
    mesh=_mesh,
    scratch_types=[
        pltpu.VMEM((EPT,), jnp.int32),
        pltpu.VMEM((N,), jnp.float32),
    ],
)
def _deg_kernel(rows_hbm, out_hbm, ridx_v, cnt_v):
    wid = _wid()
    pltpu.sync_copy(rows_hbm.at[pl.ds(wid * EPT, EPT)], ridx_v)
    zeros16 = jnp.zeros((16,), jnp.float32)
    ones16 = jnp.ones((16,), jnp.float32)

    def zero_body(i, _):
        cnt_v[pl.ds(i * 16, 16)] = zeros16
        return None

    lax.fori_loop(0, N // 16, zero_body, None)

    def add_body(i, _):
        idx = ridx_v[pl.ds(i * 16, 16)]
        plsc.addupdate_scatter(cnt_v, [idx], ones16)
        return None

    lax.fori_loop(0, EPT // 16, add_body, None)
    pltpu.sync_copy(cnt_v, out_hbm.at[wid])


@functools.partial(
    pl.kernel,
    out_type=jax.ShapeDtypeStruct((NC, N, H), jnp.float32),
    mesh=_mesh,
    scratch_types=[
        pltpu.VMEM((EW,), jnp.int32),
        pltpu.VMEM((EW,), jnp.int32),
        pltpu.VMEM((EW, H), jnp.float32),
        pltpu.VMEM_SHARED((N, H), jnp.float32),
        pltpu.SemaphoreType.DMA,
    ],
)
def _spmm_kernel(s_hbm, rows_hbm, cols_hbm, zeros_hbm, accp_hbm,
                 ridx_v, cidx_v, vals_v, acc_sh, sem):
    cid = lax.axis_index("c")
    sid = lax.axis_index("s")
    wid = sid * NC + cid
    pltpu.sync_copy(zeros_hbm, acc_sh.at[pl.ds(sid * ROWS_PER_SUB, ROWS_PER_SUB)])
    plsc.subcore_barrier()

    def body(i, _):
        base = wid * EPT + i * EW
        pltpu.sync_copy(rows_hbm.at[pl.ds(base, EW)], ridx_v)
        pltpu.sync_copy(cols_hbm.at[pl.ds(base, EW)], cidx_v)
        pltpu.async_copy(s_hbm.at[cidx_v], vals_v, sem).wait()
        pltpu.sync_copy(vals_v, acc_sh.at[ridx_v], add=True)
        return None

    lax.fori_loop(0, N_EWIN, body, None)
    plsc.subcore_barrier()
    pltpu.sync_copy(
        acc_sh.at[pl.ds(sid * ROWS_PER_SUB, ROWS_PER_SUB)],
        accp_hbm.at[cid, pl.ds(sid * ROWS_PER_SUB, ROWS_PER_SUB)],
    )


@functools.partial(
    pl.kernel,
    out_type=[
        jax.ShapeDtypeStruct((NPAIR_PAD, H), jnp.float32),
        jax.ShapeDtypeStruct((NPAIR_PAD,), jnp.int32),
    ],
    mesh=_mesh,
    scratch_types=[
        pltpu.VMEM((N,), jnp.int32),
        pltpu.VMEM((PW,), jnp.int32),
        pltpu.VMEM((PW,), jnp.int32),
        pltpu.VMEM((PW, H), jnp.float32),
        pltpu.VMEM((PW, H), jnp.float32),
        pltpu.VMEM((PW,), jnp.int32),
        pltpu.SemaphoreType.DMA,
    ],
)
def _link_kernel(z_hbm, r_hbm, c_hbm, sens_hbm, le_hbm, gs_hbm,
                 sens_v, ridx_v, cidx_v, bufr, bufc, sbuf, sem):
    wid = _wid()
    pltpu.sync_copy(sens_hbm, sens_v)

    def body(i, _):
        base = wid * PPT + i * PW
        pltpu.sync_copy(r_hbm.at[pl.ds(base, PW)], ridx_v)
        pltpu.sync_copy(c_hbm.at[pl.ds(base, PW)], cidx_v)
        cp1 = pltpu.async_copy(z_hbm.at[ridx_v], bufr, sem)
        cp2 = pltpu.async_copy(z_hbm.at[cidx_v], bufc, sem)
        cp1.wait()
        cp2.wait()

        def rowbody(p, _):
            for k in range(H // 16):
                sl = pl.ds(k * 16, 16)
                bufr[p, sl] = bufr[p, sl] * bufc[p, sl]
            return None

        lax.fori_loop(0, PW, rowbody, None)
        pltpu.sync_copy(bufr, le_hbm.at[pl.ds(base, PW)])
        for t in range(PW // 16):
            sl = pl.ds(t * 16, 16)
            rv = ridx_v[sl]
            cv = cidx_v[sl]
            sr = plsc.load_gather(sens_v, [rv])
            sc = plsc.load_gather(sens_v, [cv])
            sbuf[sl] = sr + sc
        pltpu.sync_copy(sbuf, gs_hbm.at[pl.ds(base, PW)])
        return None

    lax.fori_loop(0, N_PWIN, body, None)


def _tc1_body(deg_ref, x_ref, w1_ref, s1_ref, dinv_ref):
    ones = jnp.ones((NW, 1), jnp.float32)
    cnt = lax.dot_general(deg_ref[...], ones, (((0,), (0,)), ((), ())),
                          preferred_element_type=jnp.float32)
    dinv = lax.rsqrt(cnt + 1.0)
    s1 = dinv * jnp.dot(x_ref[...], w1_ref[...], preferred_element_type=jnp.float32)
    s1_ref[...] = s1
    dinv_ref[...] = dinv


def _tc_mid_body(accp_ref, s_ref, dinv_ref, w_ref, b_ref, out_ref):
    dinv = dinv_ref[...]
    acc = accp_ref[0] + accp_ref[1] + s_ref[...]
    h = jnp.maximum(dinv * acc + b_ref[...], 0.0)
    out_ref[...] = dinv * jnp.dot(h, w_ref[...], preferred_element_type=jnp.float32)


def _tc_final_body(accp_ref, s_ref, dinv_ref, b_ref, z_ref):
    dinv = dinv_ref[...]
    z_ref[...] = dinv * (accp_ref[0] + accp_ref[1] + s_ref[...]) + b_ref[...]


def kernel(x, edge_index, sens, neg_idx, W1, b1, W2, b2, W3, b3):
    rows_e = edge_index[0]
    cols_e = edge_index[1]
    self_loops = jnp.arange(N, dtype=jnp.int32)
    pad = jnp.zeros((NPAIR_PAD - NPAIR,), jnp.int32)
    r_all = jnp.concatenate([rows_e, self_loops, neg_idx[0], pad])
    c_all = jnp.concatenate([cols_e, self_loops, neg_idx[1], pad])
    zeros_blk = jnp.zeros((ROWS_PER_SUB, H), jnp.float32)

    deg_part = _deg_kernel(rows_e)

    s1, dinv = pl.pallas_call(
        _tc1_body,
        out_shape=[
            jax.ShapeDtypeStruct((N, H), jnp.float32),
            jax.ShapeDtypeStruct((N, 1), jnp.float32),
        ],
    )(deg_part, x, W1)

    accp1 = _spmm_kernel(s1, rows_e, cols_e, zeros_blk)
    s2 = pl.pallas_call(
        _tc_mid_body,
        out_shape=jax.ShapeDtypeStruct((N, H), jnp.float32),
    )(accp1, s1, dinv, W2, b1.reshape(1, H))

    accp2 = _spmm_kernel(s2, rows_e, cols_e, zeros_blk)
    s3 = pl.pallas_call(
        _tc_mid_body,
        out_shape=jax.ShapeDtypeStruct((N, H), jnp.float32),
    )(accp2, s2, dinv, W3, b2.reshape(1, H))

    accp3 = _spmm_kernel(s3, rows_e, cols_e, zeros_blk)
    z = pl.pallas_call(
        _tc_final_body,
        out_shape=jax.ShapeDtypeStruct((N, OUT), jnp.float32),
    )(accp3, s3, dinv, b3.reshape(1, OUT))

    le_pad, gs_pad = _link_kernel(z, r_all, c_all, sens)
    link_embeddings = le_pad[:NPAIR]
    groups_sub = gs_pad[:NPAIR]
    groups_mixed = groups_sub == 1
    labels = jnp.concatenate([jnp.ones((E_POS,), jnp.float32),
                              jnp.zeros((E_POS,), jnp.float32)])
    return link_embeddings, labels, groups_mixed, groups_sub


# speedup vs baseline: 10.5919x; 10.5919x over previous
import functools

import jax
import jax.numpy as jnp
from jax import lax
from jax.experimental import pallas as pl
from jax.experimental.pallas import tpu as pltpu
from jax.experimental.pallas import tpu_sc as plsc

N = 10000
E = 320000
D = 128
H = 64
OUT = 64
E_POS = E + N
NPAIR = 2 * E_POS

NC = 2
NS = 16
NW = NC * NS
EPT = E // NW
SUB_CHUNK = 632
LAST_CHUNK = N - (NS - 1) * SUB_CHUNK

EW = 80
N_EWIN = EPT // EW

PW = 128
PPT = 20736
N_PWIN = PPT // PW
NPAIR_PAD = NW * PPT

_mesh = plsc.VectorSubcoreMesh(core_axis_name="c", subcore_axis_name="s")
_sc_params = pltpu.CompilerParams(needs_layout_passes=False,
                                  use_tc_tiling_on_sc=False)


def _wid():
    return lax.axis_index("s") * NC + lax.axis_index("c")


@functools.partial(
    pl.kernel,
    out_type=jax.ShapeDtypeStruct((NW, N), jnp.float32),
    mesh=_mesh,
    compiler_params=_sc_params,
    scratch_types=[
        pltpu.VMEM((EPT,), jnp.int32),
        pltpu.VMEM((N,), jnp.float32),
    ],
)
def _deg_kernel(rows_hbm, out_hbm, ridx_v, cnt_v):
    wid = _wid()
    pltpu.sync_copy(rows_hbm.at[pl.ds(wid * EPT, EPT)], ridx_v)
    zeros16 = jnp.zeros((16,), jnp.float32)
    ones16 = jnp.ones((16,), jnp.float32)

    def zero_body(i, _):
        cnt_v[pl.ds(i * 16, 16)] = zeros16
        return None

    lax.fori_loop(0, N // 16, zero_body, None)

    def add_body(i, _):
        idx = ridx_v[pl.ds(i * 16, 16)]
        plsc.addupdate_scatter(cnt_v, [idx], ones16)
        return None

    lax.fori_loop(0, EPT // 16, add_body, None)
    pltpu.sync_copy(cnt_v, out_hbm.at[wid])


@functools.partial(
    pl.kernel,
    out_type=jax.ShapeDtypeStruct((NC, N, H), jnp.float32),
    mesh=_mesh,
    compiler_params=_sc_params,
    scratch_types=[
        pltpu.VMEM((EW,), jnp.int32),
        pltpu.VMEM((EW,), jnp.int32),
        pltpu.VMEM((EW, H), jnp.float32),
        pltpu.VMEM_SHARED((N, H), jnp.float32),
        pltpu.SemaphoreType.DMA,
    ],
)
def _spmm_kernel(s_hbm, rows_hbm, cols_hbm, zeros_hbm, accp_hbm,
                 ridx_v, cidx_v, vals_v, acc_sh, sem):
    cid = lax.axis_index("c")
    sid = lax.axis_index("s")
    wid = sid * NC + cid
    @pl.when(sid < NS - 1)
    def _():
        pltpu.sync_copy(zeros_hbm, acc_sh.at[pl.ds(sid * SUB_CHUNK, SUB_CHUNK)])

    @pl.when(sid == NS - 1)
    def _():
        pltpu.sync_copy(zeros_hbm.at[pl.ds(0, LAST_CHUNK)],
                        acc_sh.at[pl.ds((NS - 1) * SUB_CHUNK, LAST_CHUNK)])

    plsc.subcore_barrier()

    def body(i, _):
        base = wid * EPT + i * EW
        pltpu.sync_copy(rows_hbm.at[pl.ds(base, EW)], ridx_v)
        pltpu.sync_copy(cols_hbm.at[pl.ds(base, EW)], cidx_v)
        pltpu.async_copy(s_hbm.at[cidx_v], vals_v, sem).wait()
        pltpu.sync_copy(vals_v, acc_sh.at[ridx_v], add=True)
        return None

    lax.fori_loop(0, N_EWIN, body, None)
    plsc.subcore_barrier()

    @pl.when(sid < NS - 1)
    def _():
        pltpu.sync_copy(
            acc_sh.at[pl.ds(sid * SUB_CHUNK, SUB_CHUNK)],
            accp_hbm.at[cid, pl.ds(sid * SUB_CHUNK, SUB_CHUNK)],
        )

    @pl.when(sid == NS - 1)
    def _():
        pltpu.sync_copy(
            acc_sh.at[pl.ds((NS - 1) * SUB_CHUNK, LAST_CHUNK)],
            accp_hbm.at[cid, pl.ds((NS - 1) * SUB_CHUNK, LAST_CHUNK)],
        )


@functools.partial(
    pl.kernel,
    out_type=[
        jax.ShapeDtypeStruct((NPAIR_PAD, H), jnp.float32),
        jax.ShapeDtypeStruct((NPAIR_PAD,), jnp.int32),
    ],
    mesh=_mesh,
    compiler_params=_sc_params,
    scratch_types=[
        pltpu.VMEM((N,), jnp.int32),
        pltpu.VMEM((PW,), jnp.int32),
        pltpu.VMEM((PW,), jnp.int32),
        pltpu.VMEM((PW, H), jnp.float32),
        pltpu.VMEM((PW, H), jnp.float32),
        pltpu.VMEM((PW,), jnp.int32),
        pltpu.SemaphoreType.DMA,
    ],
)
def _link_kernel(z_hbm, r_hbm, c_hbm, sens_hbm, le_hbm, gs_hbm,
                 sens_v, ridx_v, cidx_v, bufr, bufc, sbuf, sem):
    wid = _wid()
    pltpu.sync_copy(sens_hbm, sens_v)

    def body(i, _):
        base = wid * PPT + i * PW
        pltpu.sync_copy(r_hbm.at[pl.ds(base, PW)], ridx_v)
        pltpu.sync_copy(c_hbm.at[pl.ds(base, PW)], cidx_v)
        cp1 = pltpu.async_copy(z_hbm.at[ridx_v], bufr, sem)
        cp2 = pltpu.async_copy(z_hbm.at[cidx_v], bufc, sem)
        cp1.wait()
        cp2.wait()

        def rowbody(p, _):
            for k in range(H // 16):
                sl = pl.ds(k * 16, 16)
                bufr[p, sl] = bufr[p, sl] * bufc[p, sl]
            return None

        lax.fori_loop(0, PW, rowbody, None)
        pltpu.sync_copy(bufr, le_hbm.at[pl.ds(base, PW)])
        for t in range(PW // 16):
            sl = pl.ds(t * 16, 16)
            rv = ridx_v[sl]
            cv = cidx_v[sl]
            sr = plsc.load_gather(sens_v, [rv])
            sc = plsc.load_gather(sens_v, [cv])
            sbuf[sl] = sr + sc
        pltpu.sync_copy(sbuf, gs_hbm.at[pl.ds(base, PW)])
        return None

    lax.fori_loop(0, N_PWIN, body, None)


def _tc1_body(deg_ref, x_ref, w1_ref, s1_ref, dinv_ref):
    ones = jnp.ones((NW, 1), jnp.float32)
    cnt = lax.dot_general(deg_ref[...], ones, (((0,), (0,)), ((), ())),
                          preferred_element_type=jnp.float32)
    dinv = lax.rsqrt(cnt + 1.0)
    s1 = dinv * jnp.dot(x_ref[...], w1_ref[...], preferred_element_type=jnp.float32)
    s1_ref[...] = s1
    dinv_ref[...] = dinv


def _tc_mid_body(accp_ref, s_ref, dinv_ref, w_ref, b_ref, out_ref):
    dinv = dinv_ref[...]
    acc = accp_ref[0] + accp_ref[1] + s_ref[...]
    h = jnp.maximum(dinv * acc + b_ref[...], 0.0)
    out_ref[...] = dinv * jnp.dot(h, w_ref[...], preferred_element_type=jnp.float32)


def _tc_final_body(accp_ref, s_ref, dinv_ref, b_ref, z_ref):
    dinv = dinv_ref[...]
    z_ref[...] = dinv * (accp_ref[0] + accp_ref[1] + s_ref[...]) + b_ref[...]


def kernel(x, edge_index, sens, neg_idx, W1, b1, W2, b2, W3, b3):
    rows_e = edge_index[0]
    cols_e = edge_index[1]
    self_loops = jnp.arange(N, dtype=jnp.int32)
    pad = jnp.zeros((NPAIR_PAD - NPAIR,), jnp.int32)
    r_all = jnp.concatenate([rows_e, self_loops, neg_idx[0], pad])
    c_all = jnp.concatenate([cols_e, self_loops, neg_idx[1], pad])
    zeros_blk = jnp.zeros((SUB_CHUNK, H), jnp.float32)

    deg_part = _deg_kernel(rows_e)

    s1, dinv = pl.pallas_call(
        _tc1_body,
        out_shape=[
            jax.ShapeDtypeStruct((N, H), jnp.float32),
            jax.ShapeDtypeStruct((N, 1), jnp.float32),
        ],
    )(deg_part, x, W1)

    accp1 = _spmm_kernel(s1, rows_e, cols_e, zeros_blk)
    s2 = pl.pallas_call(
        _tc_mid_body,
        out_shape=jax.ShapeDtypeStruct((N, H), jnp.float32),
    )(accp1, s1, dinv, W2, b1.reshape(1, H))

    accp2 = _spmm_kernel(s2, rows_e, cols_e, zeros_blk)
    s3 = pl.pallas_call(
        _tc_mid_body,
        out_shape=jax.ShapeDtypeStruct((N, H), jnp.float32),
    )(accp2, s2, dinv, W3, b2.reshape(1, H))

    accp3 = _spmm_kernel(s3, rows_e, cols_e, zeros_blk)
    z = pl.pallas_call(
        _tc_final_body,
        out_shape=jax.ShapeDtypeStruct((N, OUT), jnp.float32),
    )(accp3, s3, dinv, b3.reshape(1, OUT))

    le_pad, gs_pad = _link_kernel(z, r_all, c_all, sens)
    link_embeddings = le_pad[:NPAIR]
    groups_sub = gs_pad[:NPAIR]
    groups_mixed = groups_sub == 1
    labels = jnp.concatenate([jnp.ones((E_POS,), jnp.float32),
                              jnp.zeros((E_POS,), jnp.float32)])
    return link_embeddings, labels, groups_mixed, groups_sub


# speedup vs baseline: 18.5368x; 1.7501x over previous
import functools

import jax
import jax.numpy as jnp
from jax import lax
from jax.experimental import pallas as pl
from jax.experimental.pallas import tpu as pltpu
from jax.experimental.pallas import tpu_sc as plsc

N = 10000
E = 320000
D = 128
H = 64
OUT = 64
E_POS = E + N
NPAIR = 2 * E_POS

NC = 2
NS = 16
NW = NC * NS
EPT = E // NW
SUB_CHUNK = 632
LAST_CHUNK = N - (NS - 1) * SUB_CHUNK

EW = 100
WPT = EPT // EW
GW = 5
NG = WPT // GW

PW = 128
NFULL = NPAIR // PW
TAIL_OFF = NFULL * PW
TAIL = NPAIR - TAIL_OFF
NWT = (NFULL + NW - 1) // NW

_mesh = plsc.VectorSubcoreMesh(core_axis_name="c", subcore_axis_name="s")
_sc_params = pltpu.CompilerParams(needs_layout_passes=False,
                                  use_tc_tiling_on_sc=False)


def _wid():
    return lax.axis_index("s") * NC + lax.axis_index("c")


@functools.partial(
    pl.kernel,
    out_type=jax.ShapeDtypeStruct((NW, N), jnp.float32),
    mesh=_mesh,
    compiler_params=_sc_params,
    scratch_types=[
        pltpu.VMEM((EPT,), jnp.int32),
        pltpu.VMEM((N,), jnp.float32),
    ],
)
def _deg_kernel(rows_hbm, out_hbm, ridx_v, cnt_v):
    wid = _wid()
    pltpu.sync_copy(rows_hbm.at[pl.ds(wid * EPT, EPT)], ridx_v)
    zeros16 = jnp.zeros((16,), jnp.float32)
    ones16 = jnp.ones((16,), jnp.float32)

    def zero_body(i, _):
        cnt_v[pl.ds(i * 16, 16)] = zeros16
        return None

    lax.fori_loop(0, N // 16, zero_body, None)

    def add_body(i, _):
        idx = ridx_v[pl.ds(i * 16, 16)]
        plsc.addupdate_scatter(cnt_v, [idx], ones16)
        return None

    lax.fori_loop(0, EPT // 16, add_body, None)
    pltpu.sync_copy(cnt_v, out_hbm.at[wid])


@functools.partial(
    pl.kernel,
    out_type=jax.ShapeDtypeStruct((NC, N, H), jnp.float32),
    mesh=_mesh,
    compiler_params=_sc_params,
    scratch_types=[
        pltpu.VMEM((WPT, EW), jnp.int32),
        pltpu.VMEM((WPT, EW), jnp.int32),
        pltpu.VMEM((2 * GW, EW, H), jnp.float32),
        pltpu.VMEM_SHARED((N, H), jnp.float32),
        pltpu.SemaphoreType.DMA,
        pltpu.SemaphoreType.DMA,
        pltpu.SemaphoreType.DMA((2,)),
    ],
)
def _spmm_kernel(s_hbm, rows2d_hbm, cols2d_hbm, zeros_hbm, accp_hbm,
                 ridx_all, cidx_all, vals, acc_sh, isem, gsem, ssem):
    cid = lax.axis_index("c")
    sid = lax.axis_index("s")
    wid = sid * NC + cid
    ld_r = pltpu.async_copy(rows2d_hbm.at[pl.ds(wid * WPT, WPT)], ridx_all, isem)
    ld_c = pltpu.async_copy(cols2d_hbm.at[pl.ds(wid * WPT, WPT)], cidx_all, isem)

    @pl.when(sid < NS - 1)
    def _():
        pltpu.sync_copy(zeros_hbm, acc_sh.at[pl.ds(sid * SUB_CHUNK, SUB_CHUNK)])

    @pl.when(sid == NS - 1)
    def _():
        pltpu.sync_copy(zeros_hbm.at[pl.ds(0, LAST_CHUNK)],
                        acc_sh.at[pl.ds((NS - 1) * SUB_CHUNK, LAST_CHUNK)])

    plsc.subcore_barrier()
    ld_r.wait()
    ld_c.wait()

    for j in range(GW):
        pltpu.async_copy(s_hbm.at[cidx_all.at[j]], vals.at[j], gsem)

    def body(g, _):
        b = lax.rem(g, 2)
        vb = b * GW
        nvb = (1 - b) * GW
        for j in range(GW):
            w = g * GW + j
            pltpu.make_async_copy(
                s_hbm.at[cidx_all.at[w]], vals.at[vb + j], gsem).wait()
        for j in range(GW):
            w = g * GW + j
            pltpu.async_copy(vals.at[vb + j], acc_sh.at[ridx_all.at[w]],
                             ssem.at[b], add=True)

        @pl.when(g >= 1)
        def _():
            for j in range(GW):
                w = (g - 1) * GW + j
                pltpu.make_async_copy(
                    vals.at[nvb + j], acc_sh.at[ridx_all.at[w]],
                    ssem.at[1 - b]).wait()

        @pl.when(g < NG - 1)
        def _():
            for j in range(GW):
                w = (g + 1) * GW + j
                pltpu.async_copy(s_hbm.at[cidx_all.at[w]], vals.at[nvb + j],
                                 gsem)
        return None

    lax.fori_loop(0, NG, body, None)
    for j in range(GW):
        w = (NG - 1) * GW + j
        pltpu.make_async_copy(
            vals.at[GW + j], acc_sh.at[ridx_all.at[w]], ssem.at[1]).wait()
    plsc.subcore_barrier()

    @pl.when(sid < NS - 1)
    def _():
        pltpu.sync_copy(
            acc_sh.at[pl.ds(sid * SUB_CHUNK, SUB_CHUNK)],
            accp_hbm.at[cid, pl.ds(sid * SUB_CHUNK, SUB_CHUNK)],
        )

    @pl.when(sid == NS - 1)
    def _():
        pltpu.sync_copy(
            acc_sh.at[pl.ds((NS - 1) * SUB_CHUNK, LAST_CHUNK)],
            accp_hbm.at[cid, pl.ds((NS - 1) * SUB_CHUNK, LAST_CHUNK)],
        )


NB = 4
NBI = 8


@functools.partial(
    pl.kernel,
    out_type=[
        jax.ShapeDtypeStruct((NPAIR, H), jnp.float32),
        jax.ShapeDtypeStruct((NPAIR,), jnp.int32),
    ],
    mesh=_mesh,
    compiler_params=_sc_params,
    scratch_types=[
        pltpu.VMEM((N,), jnp.int32),
        pltpu.VMEM((NBI, PW), jnp.int32),
        pltpu.VMEM((NBI, PW), jnp.int32),
        pltpu.VMEM((NB, PW, H), jnp.float32),
        pltpu.VMEM((NB, PW, H), jnp.float32),
        pltpu.VMEM((NB, PW), jnp.int32),
        pltpu.VMEM((TAIL,), jnp.int32),
        pltpu.VMEM((TAIL,), jnp.int32),
        pltpu.VMEM((TAIL, H), jnp.float32),
        pltpu.VMEM((TAIL, H), jnp.float32),
        pltpu.VMEM((TAIL,), jnp.int32),
        pltpu.SemaphoreType.DMA,
        pltpu.SemaphoreType.DMA((NB,)),
        pltpu.SemaphoreType.DMA,
    ],
)
def _link_kernel(z_hbm, r_hbm, c_hbm, sens_hbm, le_hbm, gs_hbm,
                 sens_v, ridx, cidx, bufr, bufc, sbuf,
                 tir, tic, tvr, tvc, tsb, isem, gsem, osem):
    wid = _wid()
    pltpu.sync_copy(sens_hbm, sens_v)

    def win(g):
        return g * NW + wid

    def valid(g):
        return jnp.logical_and(g >= 0,
                               jnp.logical_and(g < NWT, win(g) < NFULL))

    def fire_idx(g):
        bi = lax.rem(g, NBI)
        base = win(g) * PW
        pltpu.async_copy(r_hbm.at[pl.ds(base, PW)], ridx.at[bi], isem)
        pltpu.async_copy(c_hbm.at[pl.ds(base, PW)], cidx.at[bi], isem)

    def wait_idx(g):
        bi = lax.rem(g, NBI)
        base = win(g) * PW
        pltpu.make_async_copy(r_hbm.at[pl.ds(base, PW)], ridx.at[bi], isem).wait()
        pltpu.make_async_copy(c_hbm.at[pl.ds(base, PW)], cidx.at[bi], isem).wait()

    def fire_gather(g):
        bi = lax.rem(g, NBI)
        bd = lax.rem(g, NB)
        pltpu.async_copy(z_hbm.at[ridx.at[bi]], bufr.at[bd], gsem.at[bd])
        pltpu.async_copy(z_hbm.at[cidx.at[bi]], bufc.at[bd], gsem.at[bd])

    def wait_gather(g):
        bi = lax.rem(g, NBI)
        bd = lax.rem(g, NB)
        pltpu.make_async_copy(z_hbm.at[ridx.at[bi]], bufr.at[bd],
                              gsem.at[bd]).wait()
        pltpu.make_async_copy(z_hbm.at[cidx.at[bi]], bufc.at[bd],
                              gsem.at[bd]).wait()

    def fire_out(g):
        bd = lax.rem(g, NB)
        base = win(g) * PW
        pltpu.async_copy(bufr.at[bd], le_hbm.at[pl.ds(base, PW)], osem)
        pltpu.async_copy(sbuf.at[bd], gs_hbm.at[pl.ds(base, PW)], osem)

    def wait_out(g):
        bd = lax.rem(g, NB)
        base = win(g) * PW
        pltpu.make_async_copy(bufr.at[bd], le_hbm.at[pl.ds(base, PW)], osem).wait()
        pltpu.make_async_copy(sbuf.at[bd], gs_hbm.at[pl.ds(base, PW)], osem).wait()

    def compute(g):
        bi = lax.rem(g, NBI)
        bd = lax.rem(g, NB)

        def rowbody(p, _):
            for k in range(H // 16):
                sl = pl.ds(k * 16, 16)
                bufr[bd, p, sl] = bufr[bd, p, sl] * bufc[bd, p, sl]
            return None

        lax.fori_loop(0, PW, rowbody, None)
        for t in range(PW // 16):
            sl = pl.ds(t * 16, 16)
            sr = plsc.load_gather(sens_v, [ridx[bi, sl]])
            sc = plsc.load_gather(sens_v, [cidx[bi, sl]])
            sbuf[bd, sl] = sr + sc

    def body(i, _):
        g = i - 3

        @pl.when(valid(g - 1))
        def _():
            wait_out(g - 1)

        @pl.when(valid(g + 3))
        def _():
            wait_idx(g + 3)
            fire_gather(g + 3)

        @pl.when(valid(g + 4))
        def _():
            fire_idx(g + 4)

        @pl.when(valid(g))
        def _():
            wait_gather(g)
            compute(g)
            fire_out(g)

        return None

    @pl.when(valid(0))
    def _():
        fire_idx(0)

    lax.fori_loop(0, NWT + 3, body, None)

    @pl.when(valid(NWT - 1))
    def _():
        wait_out(NWT - 1)

    @pl.when(wid == 0)
    def _():
        pltpu.sync_copy(r_hbm.at[pl.ds(TAIL_OFF, TAIL)], tir)
        pltpu.sync_copy(c_hbm.at[pl.ds(TAIL_OFF, TAIL)], tic)
        cp1 = pltpu.async_copy(z_hbm.at[tir], tvr, gsem.at[0])
        cp2 = pltpu.async_copy(z_hbm.at[tic], tvc, gsem.at[0])
        cp1.wait()
        cp2.wait()

        def trow(p, _):
            for k in range(H // 16):
                sl = pl.ds(k * 16, 16)
                tvr[p, sl] = tvr[p, sl] * tvc[p, sl]
            return None

        lax.fori_loop(0, TAIL, trow, None)
        for t in range(TAIL // 16):
            sl = pl.ds(t * 16, 16)
            sr = plsc.load_gather(sens_v, [tir[sl]])
            sc = plsc.load_gather(sens_v, [tic[sl]])
            tsb[sl] = sr + sc
        pltpu.sync_copy(tvr, le_hbm.at[pl.ds(TAIL_OFF, TAIL)])
        pltpu.sync_copy(tsb, gs_hbm.at[pl.ds(TAIL_OFF, TAIL)])


def _tc0_body(x_ref, w1_ref, v1_ref):
    v1_ref[...] = jnp.dot(x_ref[...], w1_ref[...],
                          preferred_element_type=jnp.float32)


def _tc1_body(deg_ref, v1_ref, s1_ref, dinv_ref):
    ones = jnp.ones((NW, 1), jnp.float32)
    cnt = lax.dot_general(deg_ref[...], ones, (((0,), (0,)), ((), ())),
                          preferred_element_type=jnp.float32)
    dinv = lax.rsqrt(cnt + 1.0)
    s1_ref[...] = dinv * v1_ref[...]
    dinv_ref[...] = dinv


def _tc_mid_body(accp_ref, s_ref, dinv_ref, w_ref, b_ref, out_ref):
    dinv = dinv_ref[...]
    acc = accp_ref[0] + accp_ref[1] + s_ref[...]
    h = jnp.maximum(dinv * acc + b_ref[...], 0.0)
    out_ref[...] = dinv * jnp.dot(h, w_ref[...],
                                  preferred_element_type=jnp.float32)


def _tc_final_body(accp_ref, s_ref, dinv_ref, b_ref, z_ref):
    dinv = dinv_ref[...]
    z_ref[...] = dinv * (accp_ref[0] + accp_ref[1] + s_ref[...]) + b_ref[...]


def kernel(x, edge_index, sens, neg_idx, W1, b1, W2, b2, W3, b3):
    rows_e = edge_index[0]
    cols_e = edge_index[1]
    rows2d = rows_e.reshape(E // EW, EW)
    cols2d = cols_e.reshape(E // EW, EW)
    self_loops = jnp.arange(N, dtype=jnp.int32)
    r_all = jnp.concatenate([rows_e, self_loops, neg_idx[0]])
    c_all = jnp.concatenate([cols_e, self_loops, neg_idx[1]])
    zeros_blk = jnp.zeros((SUB_CHUNK, H), jnp.float32)

    deg_part = _deg_kernel(rows_e)
    v1 = pl.pallas_call(
        _tc0_body,
        out_shape=jax.ShapeDtypeStruct((N, H), jnp.float32),
    )(x, W1)

    s1, dinv = pl.pallas_call(
        _tc1_body,
        out_shape=[
            jax.ShapeDtypeStruct((N, H), jnp.float32),
            jax.ShapeDtypeStruct((N, 1), jnp.float32),
        ],
    )(deg_part, v1)

    accp1 = _spmm_kernel(s1, rows2d, cols2d, zeros_blk)
    s2 = pl.pallas_call(
        _tc_mid_body,
        out_shape=jax.ShapeDtypeStruct((N, H), jnp.float32),
    )(accp1, s1, dinv, W2, b1.reshape(1, H))

    accp2 = _spmm_kernel(s2, rows2d, cols2d, zeros_blk)
    s3 = pl.pallas_call(
        _tc_mid_body,
        out_shape=jax.ShapeDtypeStruct((N, H), jnp.float32),
    )(accp2, s2, dinv, W3, b2.reshape(1, H))

    accp3 = _spmm_kernel(s3, rows2d, cols2d, zeros_blk)
    z = pl.pallas_call(
        _tc_final_body,
        out_shape=jax.ShapeDtypeStruct((N, OUT), jnp.float32),
    )(accp3, s3, dinv, b3.reshape(1, OUT))

    link_embeddings, groups_sub = _link_kernel(z, r_all, c_all, sens)
    groups_mixed = groups_sub == 1
    labels = jnp.concatenate([jnp.ones((E_POS,), jnp.float32),
                              jnp.zeros((E_POS,), jnp.float32)])
    return link_embeddings, labels, groups_mixed, groups_sub
